# baseline (speedup 1.0000x reference)
import functools

import jax
import jax.numpy as jnp
from jax import lax
from jax.experimental import pallas as pl
from jax.experimental.pallas import tpu as pltpu
from jax.experimental.pallas import tpu_sc as plsc

_B = 16384
_L = 20
_D = 16
_V = 1000000
_NW = 32
_BT = _B // _NW
_C = 128
_NCHUNK = _BT // _C
_NBG = _C // 16
_GPC = _C * _L // 128
_L2_NORM = 1e-4

_TR = 256
_TCOL = _TR * 128 // 16


def _tp_body(x_ref, o_ref):
    y = x_ref[...].T.reshape(_TR, 8, _D)
    o_ref[...] = jnp.concatenate([y[:, j, :] for j in range(8)], axis=-1)


_transpose_table = pl.pallas_call(
    _tp_body,
    grid=(pl.cdiv(_V * _D // 128, _TR),),
    in_specs=[pl.BlockSpec((_D, _TCOL), lambda i: (0, i))],
    out_specs=pl.BlockSpec((_TR, 128), lambda i: (i, 0)),
    out_shape=jax.ShapeDtypeStruct((_V * _D // 128, 128), jnp.float32),
)

_mesh = plsc.VectorSubcoreMesh(core_axis_name="c", subcore_axis_name="s")


@functools.partial(
    pl.kernel,
    mesh=_mesh,
    compiler_params=pltpu.CompilerParams(
        needs_layout_passes=False, use_tc_tiling_on_sc=False
    ),
    out_type=(
        jax.ShapeDtypeStruct((_B * _L,), jnp.float32),
        jax.ShapeDtypeStruct((_NW * _D,), jnp.float32),
    ),
    scratch_types=[
        pltpu.VMEM((_BT,), jnp.int32),
        pltpu.VMEM((_BT * _L,), jnp.int32),
        pltpu.VMEM((_BT, _D), jnp.float32),
        pltpu.VMEM((_C * _L, _D), jnp.float32),
        pltpu.VMEM((_C * _L,), jnp.float32),
        pltpu.VMEM((_D,), jnp.float32),
        pltpu.SemaphoreType.DMA,
    ],
)
def _sc_mf(users_f, items_f, uemb, iemb, pred_out, l2_out,
           uidx, iidx, urows, irows, predbuf, l2buf, sem):
    wid = lax.axis_index("s") * 2 + lax.axis_index("c")
    lane = jnp.arange(_D, dtype=jnp.int32)
    dsplat = [jnp.full((_D,), d, jnp.int32) for d in range(_D)]

    pltpu.async_copy(users_f.at[pl.ds(wid * _BT, _BT)], uidx, sem).wait()
    pltpu.async_copy(
        items_f.at[pl.ds(wid * _BT * _L, _BT * _L)], iidx, sem
    ).wait()
    u_handles = [
        pltpu.async_copy(
            uemb.at[uidx.at[pl.ds(g * 128, 128)]],
            urows.at[pl.ds(g * 128, 128)], sem,
        )
        for g in range(_BT // 128)
    ]
    for h in u_handles:
        h.wait()

    def chunk_body(c, accs):
        acc_u, acc_i = accs
        handles = [
            pltpu.async_copy(
                iemb.at[iidx.at[pl.ds(c * _C * _L + j * 128, 128)]],
                irows.at[pl.ds(j * 128, 128)], sem,
            )
            for j in range(_GPC)
        ]
        for h in handles:
            h.wait()

        def bg_body(bg, bg_accs):
            a_u, a_i = bg_accs
            rows_u = c * _C + bg * 16 + lane
            u_vecs = [plsc.load_gather(urows, [rows_u, dsplat[d]])
                      for d in range(_D)]
            for d in range(_D):
                a_u = a_u + u_vecs[d] * u_vecs[d]
            rows0 = (bg * 16 + lane) * _L

            def l_body(l, a_i_in):
                rows_l = rows0 + l
                acc = jnp.zeros((_D,), jnp.float32)
                for d in range(_D):
                    iv = plsc.load_gather(irows, [rows_l, dsplat[d]])
                    acc = acc + u_vecs[d] * iv
                    a_i_in = a_i_in + iv * iv
                plsc.store_scatter(predbuf, [rows_l], acc)
                return a_i_in

            a_i = lax.fori_loop(0, _L, l_body, a_i)
            return a_u, a_i

        acc_u, acc_i = lax.fori_loop(0, _NBG, bg_body, (acc_u, acc_i))
        pltpu.async_copy(
            predbuf,
            pred_out.at[pl.ds((wid * _NCHUNK + c) * _C * _L, _C * _L)],
            sem,
        ).wait()
        return acc_u, acc_i

    zero = jnp.zeros((_D,), jnp.float32)
    acc_u, acc_i = lax.fori_loop(0, _NCHUNK, chunk_body, (zero, zero))
    l2buf[...] = acc_i + jnp.float32(_L) * acc_u
    pltpu.async_copy(l2buf, l2_out.at[pl.ds(wid * _D, _D)], sem).wait()


def kernel(users, items, user_embedding, item_embedding):
    users_f = users.reshape(_B).astype(jnp.int32)
    items_f = items.astype(jnp.int32).reshape(_B * _L)
    uemb_rm = _transpose_table(user_embedding.T).reshape(_V, _D)
    iemb_rm = _transpose_table(item_embedding.T).reshape(_V, _D)
    pred_flat, l2_parts = _sc_mf(users_f, items_f, uemb_rm, iemb_rm)
    l2 = _L2_NORM * jnp.sum(l2_parts)
    return pred_flat.reshape(_B, _L), l2


# speedup vs baseline: 2.1039x; 2.1039x over previous
import functools

import jax
import jax.numpy as jnp
from jax import lax
from jax.experimental import pallas as pl
from jax.experimental.pallas import tpu as pltpu
from jax.experimental.pallas import tpu_sc as plsc

_B = 16384
_L = 20
_D = 16
_V = 1000000
_NW = 32
_BT = _B // _NW
_C = 128
_NCHUNK = _BT // _C
_NBG = _C // 16
_GPC = _C * _L // 128
_L2_NORM = 1e-4

_NBLK = _V // 128
_BPT = _NBLK // _NW + 1

_mesh = plsc.VectorSubcoreMesh(core_axis_name="c", subcore_axis_name="s")


@functools.partial(
    pl.kernel,
    mesh=_mesh,
    compiler_params=pltpu.CompilerParams(needs_layout_passes=False),
    out_type=(
        jax.ShapeDtypeStruct((_V * _D,), jnp.float32),
        jax.ShapeDtypeStruct((_V * _D,), jnp.float32),
    ),
    scratch_types=[
        pltpu.VMEM((4 * _D, 128), jnp.float32),
        pltpu.VMEM((4 * 128 * _D,), jnp.float32),
        pltpu.SemaphoreType.DMA,
        pltpu.SemaphoreType.DMA,
        pltpu.SemaphoreType.DMA,
        pltpu.SemaphoreType.DMA,
        pltpu.SemaphoreType.DMA,
        pltpu.SemaphoreType.DMA,
        pltpu.SemaphoreType.DMA,
        pltpu.SemaphoreType.DMA,
    ],
)
def _sc_relayout(uT, iT, utail, itail, urm, irm, xbuf, obuf,
                 si0, si1, si2, si3, so0, so1, so2, so3):
    wid = lax.axis_index("s") * 2 + lax.axis_index("c")
    lane16 = jnp.arange(_D, dtype=jnp.int32) * _D
    sins = [si0, si1, si2, si3]
    souts = [so0, so1, so2, so3]

    def relayout(tbl, out):
        def xslot(s):
            return xbuf.at[pl.ds(s * _D, _D), :]

        def oslot(s):
            return obuf.at[pl.ds(s * 128 * _D, 128 * _D)]

        def fire_in(k, s):
            @pl.when(k < _NBLK)
            def _():
                pltpu.async_copy(
                    tbl.at[:, pl.ds(k * 128, 128)], xslot(s), sins[s]
                )

        def hbm_dst(k):
            return out.at[pl.ds(k * 128 * _D, 128 * _D)]

        for s in range(4):
            fire_in(wid + _NW * s, s)

        def outer(jo, carry):
            for s in range(4):
                j = 4 * jo + s
                k = wid + _NW * j

                @pl.when(k < _NBLK)
                def _(j=j, k=k, s=s):
                    pltpu.make_async_copy(
                        tbl.at[:, pl.ds(k * 128, 128)], xslot(s), sins[s]
                    ).wait()

                    @pl.when(j >= 4)
                    def _():
                        pltpu.make_async_copy(
                            oslot(s), hbm_dst(k), souts[s]
                        ).wait()

                    for d in range(_D):
                        for c0 in range(8):
                            v = xbuf[s * _D + d, pl.ds(c0 * 16, 16)]
                            idx = lane16 + (s * 2048 + c0 * 256 + d)
                            plsc.store_scatter(obuf, [idx], v)
                    pltpu.async_copy(oslot(s), hbm_dst(k), souts[s])
                    fire_in(k + 4 * _NW, s)

            return carry

        lax.fori_loop(0, (_BPT + 3) // 4, outer, 0)
        for s in range(4):
            nfired = (_NBLK - wid - s * _NW + 4 * _NW - 1) // (4 * _NW)

            @pl.when(nfired > 0)
            def _(s=s):
                pltpu.make_async_copy(
                    oslot(s), hbm_dst(0), souts[s]
                ).wait()

    relayout(uT, urm)
    relayout(iT, irm)

    def copy_tail(tail, out):
        pltpu.async_copy(tail, obuf.at[pl.ds(0, 64 * _D)], si0).wait()
        pltpu.async_copy(
            obuf.at[pl.ds(0, 64 * _D)],
            out.at[pl.ds(_NBLK * 128 * _D, 64 * _D)], si0,
        ).wait()

    @pl.when(wid == 0)
    def _():
        copy_tail(utail, urm)

    @pl.when(wid == 1)
    def _():
        copy_tail(itail, irm)


@functools.partial(
    pl.kernel,
    mesh=_mesh,
    compiler_params=pltpu.CompilerParams(
        needs_layout_passes=False, use_tc_tiling_on_sc=False
    ),
    out_type=(
        jax.ShapeDtypeStruct((_B * _L,), jnp.float32),
        jax.ShapeDtypeStruct((_NW * _D,), jnp.float32),
    ),
    scratch_types=[
        pltpu.VMEM((_BT,), jnp.int32),
        pltpu.VMEM((_BT * _L,), jnp.int32),
        pltpu.VMEM((_BT, _D), jnp.float32),
        pltpu.VMEM((_C * _L, _D), jnp.float32),
        pltpu.VMEM((_C * _L,), jnp.float32),
        pltpu.VMEM((_D,), jnp.float32),
        pltpu.SemaphoreType.DMA,
    ],
)
def _sc_mf(users_f, items_f, uemb, iemb, pred_out, l2_out,
           uidx, iidx, urows, irows, predbuf, l2buf, sem):
    wid = lax.axis_index("s") * 2 + lax.axis_index("c")
    lane = jnp.arange(_D, dtype=jnp.int32)
    dsplat = [jnp.full((_D,), d, jnp.int32) for d in range(_D)]

    pltpu.async_copy(users_f.at[pl.ds(wid * _BT, _BT)], uidx, sem).wait()
    pltpu.async_copy(
        items_f.at[pl.ds(wid * _BT * _L, _BT * _L)], iidx, sem
    ).wait()
    u_handles = [
        pltpu.async_copy(
            uemb.at[uidx.at[pl.ds(g * 128, 128)]],
            urows.at[pl.ds(g * 128, 128)], sem,
        )
        for g in range(_BT // 128)
    ]
    for h in u_handles:
        h.wait()

    def chunk_body(c, accs):
        acc_u, acc_i = accs
        handles = [
            pltpu.async_copy(
                iemb.at[iidx.at[pl.ds(c * _C * _L + j * 128, 128)]],
                irows.at[pl.ds(j * 128, 128)], sem,
            )
            for j in range(_GPC)
        ]
        for h in handles:
            h.wait()

        def bg_body(bg, bg_accs):
            a_u, a_i = bg_accs
            rows_u = c * _C + bg * 16 + lane
            u_vecs = [plsc.load_gather(urows, [rows_u, dsplat[d]])
                      for d in range(_D)]
            for d in range(_D):
                a_u = a_u + u_vecs[d] * u_vecs[d]
            rows0 = (bg * 16 + lane) * _L

            def l_body(l, a_i_in):
                rows_l = rows0 + l
                acc = jnp.zeros((_D,), jnp.float32)
                for d in range(_D):
                    iv = plsc.load_gather(irows, [rows_l, dsplat[d]])
                    acc = acc + u_vecs[d] * iv
                    a_i_in = a_i_in + iv * iv
                plsc.store_scatter(predbuf, [rows_l], acc)
                return a_i_in

            a_i = lax.fori_loop(0, _L, l_body, a_i)
            return a_u, a_i

        acc_u, acc_i = lax.fori_loop(0, _NBG, bg_body, (acc_u, acc_i))
        pltpu.async_copy(
            predbuf,
            pred_out.at[pl.ds((wid * _NCHUNK + c) * _C * _L, _C * _L)],
            sem,
        ).wait()
        return acc_u, acc_i

    zero = jnp.zeros((_D,), jnp.float32)
    acc_u, acc_i = lax.fori_loop(0, _NCHUNK, chunk_body, (zero, zero))
    l2buf[...] = acc_i + jnp.float32(_L) * acc_u
    pltpu.async_copy(l2buf, l2_out.at[pl.ds(wid * _D, _D)], sem).wait()


def kernel(users, items, user_embedding, item_embedding):
    users_f = users.reshape(_B).astype(jnp.int32)
    items_f = items.astype(jnp.int32).reshape(_B * _L)
    utail = user_embedding[_NBLK * 128:, :].reshape(64 * _D)
    itail = item_embedding[_NBLK * 128:, :].reshape(64 * _D)
    urm_f, irm_f = _sc_relayout(user_embedding.T, item_embedding.T,
                                utail, itail)
    uemb_rm = urm_f.reshape(_V, _D)
    iemb_rm = irm_f.reshape(_V, _D)
    pred_flat, l2_parts = _sc_mf(users_f, items_f, uemb_rm, iemb_rm)
    l2 = _L2_NORM * jnp.sum(l2_parts)
    return pred_flat.reshape(_B, _L), l2


# speedup vs baseline: 2.1141x; 1.0048x over previous
import functools

import jax
import jax.numpy as jnp
from jax import lax
from jax.experimental import pallas as pl
from jax.experimental.pallas import tpu as pltpu
from jax.experimental.pallas import tpu_sc as plsc

_B = 16384
_L = 20
_D = 16
_V = 1000000
_NW = 32
_BT = _B // _NW
_C = 128
_NCHUNK = _BT // _C
_NBG = _C // 16
_GPC = _C * _L // 128
_L2_NORM = 1e-4

_NBLK = _V // 128
_BPT = _NBLK // _NW + 1

_mesh = plsc.VectorSubcoreMesh(core_axis_name="c", subcore_axis_name="s")


@functools.partial(
    pl.kernel,
    mesh=_mesh,
    compiler_params=pltpu.CompilerParams(needs_layout_passes=False),
    out_type=(
        jax.ShapeDtypeStruct((_V * _D,), jnp.float32),
        jax.ShapeDtypeStruct((_V * _D,), jnp.float32),
    ),
    scratch_types=[
        pltpu.VMEM((3 * _D, 128), jnp.float32),
        pltpu.VMEM((3 * 128 * _D,), jnp.float32),
        pltpu.SemaphoreType.DMA,
        pltpu.SemaphoreType.DMA,
        pltpu.SemaphoreType.DMA,
        pltpu.SemaphoreType.DMA,
        pltpu.SemaphoreType.DMA,
        pltpu.SemaphoreType.DMA,
    ],
)
def _sc_relayout(uT, iT, utail, itail, urm, irm, xbuf, obuf,
                 si0, si1, si2, so0, so1, so2):
    wid = lax.axis_index("s") * 2 + lax.axis_index("c")
    lane16 = jnp.arange(_D, dtype=jnp.int32) * _D
    sins = [si0, si1, si2]
    souts = [so0, so1, so2]

    def relayout(tbl, out):
        def xslot(s):
            return xbuf.at[pl.ds(s * _D, _D), :]

        def oslot(s):
            return obuf.at[pl.ds(s * 128 * _D, 128 * _D)]

        def fire_in(k, s):
            @pl.when(k < _NBLK)
            def _():
                pltpu.async_copy(
                    tbl.at[:, pl.ds(k * 128, 128)], xslot(s), sins[s]
                )

        def hbm_dst(k):
            return out.at[pl.ds(k * 128 * _D, 128 * _D)]

        for s in range(3):
            fire_in(wid + _NW * s, s)

        def outer(jo, carry):
            for s in range(3):
                j = 3 * jo + s
                k = wid + _NW * j

                @pl.when(k < _NBLK)
                def _(j=j, k=k, s=s):
                    pltpu.make_async_copy(
                        tbl.at[:, pl.ds(k * 128, 128)], xslot(s), sins[s]
                    ).wait()

                    @pl.when(j >= 3)
                    def _():
                        pltpu.make_async_copy(
                            oslot(s), hbm_dst(k), souts[s]
                        ).wait()

                    for d in range(_D):
                        for c0 in range(8):
                            v = xbuf[s * _D + d, pl.ds(c0 * 16, 16)]
                            idx = lane16 + (s * 2048 + c0 * 256 + d)
                            plsc.store_scatter(obuf, [idx], v)
                    pltpu.async_copy(oslot(s), hbm_dst(k), souts[s])
                    fire_in(k + 3 * _NW, s)

            return carry

        lax.fori_loop(0, (_BPT + 2) // 3, outer, 0)
        for s in range(3):
            nfired = (_NBLK - wid - s * _NW + 3 * _NW - 1) // (3 * _NW)

            @pl.when(nfired > 0)
            def _(s=s):
                pltpu.make_async_copy(
                    oslot(s), hbm_dst(0), souts[s]
                ).wait()

    relayout(uT, urm)
    relayout(iT, irm)

    def copy_tail(tail, out):
        pltpu.async_copy(tail, obuf.at[pl.ds(0, 64 * _D)], si0).wait()
        pltpu.async_copy(
            obuf.at[pl.ds(0, 64 * _D)],
            out.at[pl.ds(_NBLK * 128 * _D, 64 * _D)], si0,
        ).wait()

    @pl.when(wid == 0)
    def _():
        copy_tail(utail, urm)

    @pl.when(wid == 1)
    def _():
        copy_tail(itail, irm)


@functools.partial(
    pl.kernel,
    mesh=_mesh,
    compiler_params=pltpu.CompilerParams(
        needs_layout_passes=False, use_tc_tiling_on_sc=False
    ),
    out_type=(
        jax.ShapeDtypeStruct((_B * _L,), jnp.float32),
        jax.ShapeDtypeStruct((_NW * _D,), jnp.float32),
    ),
    scratch_types=[
        pltpu.VMEM((_BT,), jnp.int32),
        pltpu.VMEM((_BT * _L,), jnp.int32),
        pltpu.VMEM((_BT, _D), jnp.float32),
        pltpu.VMEM((_C * _L, _D), jnp.float32),
        pltpu.VMEM((_C * _L,), jnp.float32),
        pltpu.VMEM((_D,), jnp.float32),
        pltpu.SemaphoreType.DMA,
    ],
)
def _sc_mf(users_f, items_f, uemb, iemb, pred_out, l2_out,
           uidx, iidx, urows, irows, predbuf, l2buf, sem):
    wid = lax.axis_index("s") * 2 + lax.axis_index("c")
    lane = jnp.arange(_D, dtype=jnp.int32)
    dsplat = [jnp.full((_D,), d, jnp.int32) for d in range(_D)]

    pltpu.async_copy(users_f.at[pl.ds(wid * _BT, _BT)], uidx, sem).wait()
    pltpu.async_copy(
        items_f.at[pl.ds(wid * _BT * _L, _BT * _L)], iidx, sem
    ).wait()
    u_handles = [
        pltpu.async_copy(
            uemb.at[uidx.at[pl.ds(g * 128, 128)]],
            urows.at[pl.ds(g * 128, 128)], sem,
        )
        for g in range(_BT // 128)
    ]
    for h in u_handles:
        h.wait()

    def chunk_body(c, accs):
        acc_u, acc_i = accs
        handles = [
            pltpu.async_copy(
                iemb.at[iidx.at[pl.ds(c * _C * _L + j * 128, 128)]],
                irows.at[pl.ds(j * 128, 128)], sem,
            )
            for j in range(_GPC)
        ]
        for h in handles:
            h.wait()

        def bg_body(bg, bg_accs):
            a_u, a_i = bg_accs
            rows_u = c * _C + bg * 16 + lane
            u_vecs = [plsc.load_gather(urows, [rows_u, dsplat[d]])
                      for d in range(_D)]
            for d in range(_D):
                a_u = a_u + u_vecs[d] * u_vecs[d]
            rows0 = (bg * 16 + lane) * _L

            def l_body(l, a_i_in):
                rows_l = rows0 + l
                acc = jnp.zeros((_D,), jnp.float32)
                for d in range(_D):
                    iv = plsc.load_gather(irows, [rows_l, dsplat[d]])
                    acc = acc + u_vecs[d] * iv
                    a_i_in = a_i_in + iv * iv
                plsc.store_scatter(predbuf, [rows_l], acc)
                return a_i_in

            a_i = lax.fori_loop(0, _L, l_body, a_i)
            return a_u, a_i

        acc_u, acc_i = lax.fori_loop(0, _NBG, bg_body, (acc_u, acc_i))
        pltpu.async_copy(
            predbuf,
            pred_out.at[pl.ds((wid * _NCHUNK + c) * _C * _L, _C * _L)],
            sem,
        ).wait()
        return acc_u, acc_i

    zero = jnp.zeros((_D,), jnp.float32)
    acc_u, acc_i = lax.fori_loop(0, _NCHUNK, chunk_body, (zero, zero))
    l2buf[...] = acc_i + jnp.float32(_L) * acc_u
    pltpu.async_copy(l2buf, l2_out.at[pl.ds(wid * _D, _D)], sem).wait()


def kernel(users, items, user_embedding, item_embedding):
    users_f = users.reshape(_B).astype(jnp.int32)
    items_f = items.astype(jnp.int32).reshape(_B * _L)
    utail = user_embedding[_NBLK * 128:, :].reshape(64 * _D)
    itail = item_embedding[_NBLK * 128:, :].reshape(64 * _D)
    urm_f, irm_f = _sc_relayout(user_embedding.T, item_embedding.T,
                                utail, itail)
    uemb_rm = urm_f.reshape(_V, _D)
    iemb_rm = irm_f.reshape(_V, _D)
    pred_flat, l2_parts = _sc_mf(users_f, items_f, uemb_rm, iemb_rm)
    l2 = _L2_NORM * jnp.sum(l2_parts)
    return pred_flat.reshape(_B, _L), l2


# speedup vs baseline: 2.1462x; 1.0152x over previous
import functools

import jax
import jax.numpy as jnp
from jax import lax
from jax.experimental import pallas as pl
from jax.experimental.pallas import tpu as pltpu
from jax.experimental.pallas import tpu_sc as plsc

_B = 16384
_L = 20
_D = 16
_V = 1000000
_NW = 32
_BT = _B // _NW
_C = 128
_NCHUNK = _BT // _C
_NBG = _C // 16
_GPC = _C * _L // 128
_L2_NORM = 1e-4

_NBLK = _V // 128
_BPT = _NBLK // _NW + 1

_mesh = plsc.VectorSubcoreMesh(core_axis_name="c", subcore_axis_name="s")


@functools.partial(
    pl.kernel,
    mesh=_mesh,
    compiler_params=pltpu.CompilerParams(needs_layout_passes=False),
    out_type=(
        jax.ShapeDtypeStruct((_V * _D,), jnp.float32),
        jax.ShapeDtypeStruct((_V * _D,), jnp.float32),
    ),
    scratch_types=[
        pltpu.VMEM((3 * _D, 128), jnp.float32),
        pltpu.VMEM((3 * 128 * _D,), jnp.float32),
        pltpu.SemaphoreType.DMA,
        pltpu.SemaphoreType.DMA,
        pltpu.SemaphoreType.DMA,
        pltpu.SemaphoreType.DMA,
        pltpu.SemaphoreType.DMA,
        pltpu.SemaphoreType.DMA,
    ],
)
def _sc_relayout(uT, iT, utail, itail, urm, irm, xbuf, obuf,
                 si0, si1, si2, so0, so1, so2):
    wid = lax.axis_index("s") * 2 + lax.axis_index("c")
    lane16 = jnp.arange(_D, dtype=jnp.int32) * _D
    sins = [si0, si1, si2]
    souts = [so0, so1, so2]

    def relayout(tbl, out):
        def xslot(s):
            return xbuf.at[pl.ds(s * _D, _D), :]

        def oslot(s):
            return obuf.at[pl.ds(s * 128 * _D, 128 * _D)]

        def fire_in(k, s):
            @pl.when(k < _NBLK)
            def _():
                pltpu.async_copy(
                    tbl.at[:, pl.ds(k * 128, 128)], xslot(s), sins[s]
                )

        def hbm_dst(k):
            return out.at[pl.ds(k * 128 * _D, 128 * _D)]

        for s in range(3):
            fire_in(wid + _NW * s, s)

        def outer(jo, carry):
            for s in range(3):
                j = 3 * jo + s
                k = wid + _NW * j

                @pl.when(k < _NBLK)
                def _(j=j, k=k, s=s):
                    pltpu.make_async_copy(
                        tbl.at[:, pl.ds(k * 128, 128)], xslot(s), sins[s]
                    ).wait()

                    @pl.when(j >= 3)
                    def _():
                        pltpu.make_async_copy(
                            oslot(s), hbm_dst(k), souts[s]
                        ).wait()

                    for d in range(_D):
                        for c0 in range(8):
                            v = xbuf[s * _D + d, pl.ds(c0 * 16, 16)]
                            idx = lane16 + (s * 2048 + c0 * 256 + d)
                            plsc.store_scatter(obuf, [idx], v)
                    pltpu.async_copy(oslot(s), hbm_dst(k), souts[s])
                    fire_in(k + 3 * _NW, s)

            return carry

        lax.fori_loop(0, (_BPT + 2) // 3, outer, 0)
        for s in range(3):
            nfired = (_NBLK - wid - s * _NW + 3 * _NW - 1) // (3 * _NW)

            @pl.when(nfired > 0)
            def _(s=s):
                pltpu.make_async_copy(
                    oslot(s), hbm_dst(0), souts[s]
                ).wait()

    relayout(uT, urm)
    relayout(iT, irm)

    def copy_tail(tail, out):
        pltpu.async_copy(tail, obuf.at[pl.ds(0, 64 * _D)], si0).wait()
        pltpu.async_copy(
            obuf.at[pl.ds(0, 64 * _D)],
            out.at[pl.ds(_NBLK * 128 * _D, 64 * _D)], si0,
        ).wait()

    @pl.when(wid == 0)
    def _():
        copy_tail(utail, urm)

    @pl.when(wid == 1)
    def _():
        copy_tail(itail, irm)


@functools.partial(
    pl.kernel,
    mesh=_mesh,
    compiler_params=pltpu.CompilerParams(
        needs_layout_passes=False, use_tc_tiling_on_sc=False
    ),
    out_type=(
        jax.ShapeDtypeStruct((_B * _L,), jnp.float32),
        jax.ShapeDtypeStruct((_NW * _D,), jnp.float32),
    ),
    scratch_types=[
        pltpu.VMEM((_BT,), jnp.int32),
        pltpu.VMEM((_BT * _L,), jnp.int32),
        pltpu.VMEM((_BT, _D), jnp.float32),
        pltpu.VMEM((2 * _C * _L, _D), jnp.float32),
        pltpu.VMEM((_C * _L,), jnp.float32),
        pltpu.VMEM((_D,), jnp.float32),
        pltpu.SemaphoreType.DMA,
        pltpu.SemaphoreType.DMA,
        pltpu.SemaphoreType.DMA,
    ],
)
def _sc_mf(users_f, items_f, uemb, iemb, pred_out, l2_out,
           uidx, iidx, urows, irows, predbuf, l2buf, sem, gs0, gs1):
    wid = lax.axis_index("s") * 2 + lax.axis_index("c")
    lane = jnp.arange(_D, dtype=jnp.int32)
    dsplat = [jnp.full((_D,), d, jnp.int32) for d in range(_D)]

    pltpu.async_copy(users_f.at[pl.ds(wid * _BT, _BT)], uidx, sem).wait()
    pltpu.async_copy(
        items_f.at[pl.ds(wid * _BT * _L, _BT * _L)], iidx, sem
    ).wait()
    u_handles = [
        pltpu.async_copy(
            uemb.at[uidx.at[pl.ds(g * 128, 128)]],
            urows.at[pl.ds(g * 128, 128)], sem,
        )
        for g in range(_BT // 128)
    ]
    for h in u_handles:
        h.wait()

    gsems = [gs0, gs1]

    def fire_chunk(c):
        s = c % 2
        return [
            pltpu.async_copy(
                iemb.at[iidx.at[pl.ds(c * _C * _L + j * 128, 128)]],
                irows.at[pl.ds((s * _GPC + j) * 128, 128)], gsems[s],
            )
            for j in range(_GPC)
        ]

    def chunk_body(c, accs, handles, off):
        acc_u, acc_i = accs
        for h in handles:
            h.wait()

        def bg_body(bg, bg_accs):
            a_u, a_i = bg_accs
            rows_u = c * _C + bg * 16 + lane
            u_vecs = [plsc.load_gather(urows, [rows_u, dsplat[d]])
                      for d in range(_D)]
            for d in range(_D):
                a_u = a_u + u_vecs[d] * u_vecs[d]
            rows0 = (bg * 16 + lane) * _L

            def l_body(l, a_i_in):
                rows_l = rows0 + l
                acc = jnp.zeros((_D,), jnp.float32)
                for d in range(_D):
                    iv = plsc.load_gather(irows, [rows_l + off, dsplat[d]])
                    acc = acc + u_vecs[d] * iv
                    a_i_in = a_i_in + iv * iv
                plsc.store_scatter(predbuf, [rows_l], acc)
                return a_i_in

            a_i = lax.fori_loop(0, _L, l_body, a_i)
            return a_u, a_i

        acc_u, acc_i = lax.fori_loop(0, _NBG, bg_body, (acc_u, acc_i))
        pltpu.async_copy(
            predbuf,
            pred_out.at[pl.ds((wid * _NCHUNK + c) * _C * _L, _C * _L)],
            sem,
        ).wait()
        return acc_u, acc_i

    zero = jnp.zeros((_D,), jnp.float32)
    accs = (zero, zero)
    handles = fire_chunk(0)
    for c in range(_NCHUNK):
        nxt = fire_chunk(c + 1) if c + 1 < _NCHUNK else []
        accs = chunk_body(c, accs, handles, (c % 2) * _C * _L)
        handles = nxt
    acc_u, acc_i = accs
    l2buf[...] = acc_i + jnp.float32(_L) * acc_u
    pltpu.async_copy(l2buf, l2_out.at[pl.ds(wid * _D, _D)], sem).wait()


def kernel(users, items, user_embedding, item_embedding):
    users_f = users.reshape(_B).astype(jnp.int32)
    items_f = items.astype(jnp.int32).reshape(_B * _L)
    utail = user_embedding[_NBLK * 128:, :].reshape(64 * _D)
    itail = item_embedding[_NBLK * 128:, :].reshape(64 * _D)
    urm_f, irm_f = _sc_relayout(user_embedding.T, item_embedding.T,
                                utail, itail)
    uemb_rm = urm_f.reshape(_V, _D)
    iemb_rm = irm_f.reshape(_V, _D)
    pred_flat, l2_parts = _sc_mf(users_f, items_f, uemb_rm, iemb_rm)
    l2 = _L2_NORM * jnp.sum(l2_parts)
    return pred_flat.reshape(_B, _L), l2
